# Initial kernel scaffold; baseline (speedup 1.0000x reference)
import jax
import jax.numpy as jnp
from jax.experimental import pallas as pl


def kernel(lengths, nodes_output, Wq):
    raise NotImplementedError("write your pallas kernel here")


# speedup vs baseline: 6.0128x; 6.0128x over previous
import functools

import jax
import jax.numpy as jnp
from jax.experimental import pallas as pl


def _agg_kernel(len_ref, x_ref, wq_ref, out_ref, *, t_chunk: int):
    kt = pl.program_id(1)

    @pl.when(kt == 0)
    def _init():
        out_ref[...] = jnp.zeros_like(out_ref)

    xb = x_ref[0]
    w = jax.lax.dot_general(
        xb, wq_ref[...], (((1,), (0,)), ((), ())),
        preferred_element_type=jnp.float32)

    t0 = kt * t_chunk
    t_idx = jax.lax.broadcasted_iota(jnp.int32, (1, t_chunk), 1) + t0
    mask = (t_idx < len_ref[...]).astype(jnp.float32)
    a = mask * w.reshape(1, t_chunk)

    acc = jax.lax.dot_general(
        a, xb, (((1,), (0,)), ((), ())),
        preferred_element_type=jnp.float32)
    out_ref[0] += acc


def kernel(lengths, nodes_output, Wq):
    B, T, D = nodes_output.shape
    n_len = lengths.shape[0]
    t_chunk = 512
    lens = jnp.asarray(lengths, dtype=jnp.int32).reshape(n_len, 1)

    grid = (B, T // t_chunk)
    out = pl.pallas_call(
        functools.partial(_agg_kernel, t_chunk=t_chunk),
        grid=grid,
        in_specs=[
            pl.BlockSpec((n_len, 1), lambda b, kt: (0, 0)),
            pl.BlockSpec((1, t_chunk, D), lambda b, kt: (b, kt, 0)),
            pl.BlockSpec((D, 1), lambda b, kt: (0, 0)),
        ],
        out_specs=pl.BlockSpec((1, n_len, D), lambda b, kt: (b, 0, 0)),
        out_shape=jax.ShapeDtypeStruct((B, n_len, D), jnp.float32),
    )(lens, nodes_output, Wq)
    return out.transpose(1, 0, 2).reshape(n_len * B, D)


# speedup vs baseline: 9.4205x; 1.5667x over previous
import functools

import jax
import jax.numpy as jnp
from jax.experimental import pallas as pl


def _agg_kernel(len_ref, x_ref, wq_ref, out_ref, *, t_chunk: int):
    kt = pl.program_id(1)

    @pl.when(kt == 0)
    def _init():
        out_ref[...] = jnp.zeros_like(out_ref)

    xb = x_ref[0]
    d = xb.shape[1]
    w = jnp.sum(xb * wq_ref[...].reshape(1, d), axis=1)

    t0 = kt * t_chunk
    t_idx = jax.lax.broadcasted_iota(jnp.int32, (1, t_chunk), 1) + t0
    mask = (t_idx < len_ref[...]).astype(jnp.float32)
    a = mask * w.reshape(1, t_chunk)

    acc = jax.lax.dot_general(
        a, xb, (((1,), (0,)), ((), ())),
        preferred_element_type=jnp.float32)
    out_ref[0] += acc


def kernel(lengths, nodes_output, Wq):
    B, T, D = nodes_output.shape
    n_len = lengths.shape[0]
    t_chunk = 1024
    lens = jnp.asarray(lengths, dtype=jnp.int32).reshape(n_len, 1)

    grid = (B, T // t_chunk)
    out = pl.pallas_call(
        functools.partial(_agg_kernel, t_chunk=t_chunk),
        grid=grid,
        in_specs=[
            pl.BlockSpec((n_len, 1), lambda b, kt: (0, 0)),
            pl.BlockSpec((1, t_chunk, D), lambda b, kt: (b, kt, 0)),
            pl.BlockSpec((D, 1), lambda b, kt: (0, 0)),
        ],
        out_specs=pl.BlockSpec((1, n_len, D), lambda b, kt: (b, 0, 0)),
        out_shape=jax.ShapeDtypeStruct((B, n_len, D), jnp.float32),
    )(lens, nodes_output, Wq)
    return out.transpose(1, 0, 2).reshape(n_len * B, D)
